# Initial kernel scaffold; baseline (speedup 1.0000x reference)
import jax
import jax.numpy as jnp
from jax.experimental import pallas as pl


def kernel(token_ids, E, P_w):
    raise NotImplementedError("write your pallas kernel here")


# speedup vs baseline: 1.3915x; 1.3915x over previous
import functools

import jax
import jax.numpy as jnp
from jax import lax
from jax.experimental import pallas as pl
from jax.experimental.pallas import tpu as pltpu
from jax.experimental.pallas import tpu_sc as plsc

_NC = 2
_NS = 16
_NW = _NC * _NS

_G = 128
_K = 4


def _sc_gather(ids_2d, E):
    n_chunks, g = ids_2d.shape
    assert g == _G
    v, d = E.shape
    assert n_chunks % (_NW * _K) == 0
    cpw = n_chunks // _NW
    n_blk = cpw // _K

    mesh = plsc.VectorSubcoreMesh(
        core_axis_name="c", subcore_axis_name="s",
        num_cores=_NC, num_subcores=_NS)

    @functools.partial(
        pl.kernel,
        out_type=jax.ShapeDtypeStruct((n_chunks, _G, d), jnp.float32),
        mesh=mesh,
        scratch_types=[
            pltpu.VMEM((cpw, _G), jnp.int32),
            pltpu.VMEM((_K, _G, d), jnp.float32),
            pltpu.SemaphoreType.DMA,
        ],
        compiler_params=pltpu.CompilerParams(use_tc_tiling_on_sc=False),
    )
    def gather_kernel(ids_hbm, table_hbm, emb_hbm, idx_v, rows_v, sem):
        wid = lax.axis_index("s") * _NC + lax.axis_index("c")
        chunk_base = wid * cpw
        pltpu.sync_copy(ids_hbm.at[pl.ds(chunk_base, cpw)], idx_v)

        def body(blk, carry):
            descs = []
            for k in range(_K):
                descs.append(pltpu.async_copy(
                    table_hbm.at[idx_v.at[blk * _K + k]], rows_v.at[k], sem))
            for dsc in descs:
                dsc.wait()
            pltpu.sync_copy(
                rows_v, emb_hbm.at[pl.ds(chunk_base + blk * _K, _K)])
            return carry

        lax.fori_loop(0, n_blk, body, 0)

    return gather_kernel(ids_2d, E)


def _tc_project(emb, p_wt, rm=1024):
    n, d = emb.shape
    m = p_wt.shape[1]
    assert n % rm == 0

    def mm_body(emb_ref, w_ref, out_ref):
        out_ref[...] = jnp.dot(emb_ref[...], w_ref[...],
                               preferred_element_type=jnp.float32)

    return pl.pallas_call(
        mm_body,
        grid=(n // rm,),
        in_specs=[
            pl.BlockSpec((rm, d), lambda i: (i, 0)),
            pl.BlockSpec((d, m), lambda i: (0, 0)),
        ],
        out_specs=pl.BlockSpec((rm, m), lambda i: (i, 0)),
        out_shape=jax.ShapeDtypeStruct((n, m), jnp.float32),
    )(emb, p_wt)


def kernel(token_ids, E, P_w):
    b, l = token_ids.shape
    v, d = E.shape
    m = P_w.shape[0]
    n = b * l

    ids_2d = token_ids.reshape(n // _G, _G).astype(jnp.int32)
    emb = _sc_gather(ids_2d, E).reshape(n, d)
    out = _tc_project(emb, P_w.T)
    return out.reshape(b, l, m)
